# Initial kernel scaffold; baseline (speedup 1.0000x reference)
import jax
import jax.numpy as jnp
from jax.experimental import pallas as pl


def kernel(anchor_feature, pair_feature, memory_bank, negatives, membank_idx):
    raise NotImplementedError("write your pallas kernel here")


# speedup vs baseline: 1.2772x; 1.2772x over previous
import functools

import jax
import jax.numpy as jnp
from jax import lax
from jax.experimental import pallas as pl
from jax.experimental.pallas import tpu as pltpu
from jax.experimental.pallas import tpu_sc as plsc

B = 1024
D = 128
N_ROWS = 100000
KP1 = 512
T = 0.07

NW = 32
B_PER_W = B // NW
ROWS_PER_W = N_ROWS // NW
COPY_CHUNK = 125
N_COPY_CHUNKS = ROWS_PER_W // COPY_CHUNK
GCHUNK = 128
N_GCHUNKS = KP1 // GCHUNK
L = 16
IDX_BITS = 17


def _sc_body(anchor_hbm, pair_hbm, mb_hbm, neg_hbm, mbidx_hbm,
             e_hbm, psum_hbm, newmb_hbm,
             anch_v, pair_v, idx_v, rows_v, sums_v, mbidx_v,
             hits_s, cbuf_v, psv_v, sem):
    c = lax.axis_index("c")
    s = lax.axis_index("s")
    w = s * 2 + c
    base_b = w * B_PER_W

    pltpu.sync_copy(anchor_hbm.at[pl.ds(base_b, B_PER_W)], anch_v)
    pltpu.sync_copy(pair_hbm.at[pl.ds(base_b, B_PER_W)], pair_v)
    pltpu.sync_copy(mbidx_hbm, mbidx_v)

    iota = lax.iota(jnp.int32, L)

    def b_loop(bl, psum):
        bg = base_b + bl
        cps = []
        for ci in range(N_GCHUNKS):
            pltpu.sync_copy(neg_hbm.at[bg, pl.ds(ci * GCHUNK, GCHUNK)],
                            idx_v.at[ci])
            cps.append(pltpu.async_copy(
                mb_hbm.at[idx_v.at[ci]],
                rows_v.at[pl.ds(ci * GCHUNK, GCHUNK)], sem))
        for cp in cps:
            cp.wait()
        for j in range(D // L):
            rows_v[0, pl.ds(L * j, L)] = pair_v[bl, pl.ds(L * j, L)]

        a = [anch_v[bl, pl.ds(L * j, L)] for j in range(D // L)]

        def g_loop(g, psum):
            row_idx = iota + g * L
            acc = jnp.zeros((L,), jnp.float32)
            for j in range(D // L):
                for r in range(L):
                    d = L * j + r
                    ab = a[j].at[jnp.full((L,), r, jnp.int32)].get(
                        mode="promise_in_bounds")
                    col = jnp.full((L,), d, jnp.int32)
                    acc = acc + plsc.load_gather(rows_v, [row_idx, col]) * ab
            e = jnp.exp(acc * (1.0 / T))
            sums_v[pl.ds(g * L, L)] = e
            return psum + e

        psum = lax.fori_loop(0, KP1 // L, g_loop, psum)
        pltpu.sync_copy(sums_v, e_hbm.at[bg])
        return psum

    psum = lax.fori_loop(0, B_PER_W, b_loop, jnp.zeros((L,), jnp.float32))
    psv_v[...] = psum
    pltpu.sync_copy(psv_v, psum_hbm.at[w])

    lo = w * ROWS_PER_W

    def scan_b(t, cnt):
        vec = mbidx_v[pl.ds(L * t, L)]
        for u in range(L):
            idx = vec[u]
            b = L * t + u
            hit = (idx >= lo) & (idx < lo + ROWS_PER_W)

            @pl.when(hit)
            def _(cnt=cnt, b=b, idx=idx):
                hits_s[cnt] = (b << IDX_BITS) | idx

            cnt = cnt + jnp.where(hit, 1, 0)
        return cnt

    cnt = lax.fori_loop(0, B // L, scan_b, 0)

    def chunk_loop(ch, _):
        start = lo + ch * COPY_CHUNK
        pltpu.sync_copy(mb_hbm.at[pl.ds(start, COPY_CHUNK)], cbuf_v)

        def hit_loop(j, _):
            v = hits_s[j]
            hb = v >> IDX_BITS
            hi = v & ((1 << IDX_BITS) - 1)

            @pl.when((hi >= start) & (hi < start + COPY_CHUNK))
            def _():
                pltpu.sync_copy(anchor_hbm.at[hb], cbuf_v.at[hi - start])

            return 0

        lax.fori_loop(0, cnt, hit_loop, 0)
        pltpu.sync_copy(cbuf_v, newmb_hbm.at[pl.ds(start, COPY_CHUNK)])
        return 0

    lax.fori_loop(0, N_COPY_CHUNKS, chunk_loop, 0)


_sc_call = functools.partial(
    pl.kernel,
    out_type=(
        jax.ShapeDtypeStruct((B, KP1), jnp.float32),
        jax.ShapeDtypeStruct((NW, L), jnp.float32),
        jax.ShapeDtypeStruct((N_ROWS, D), jnp.float32),
    ),
    mesh=plsc.VectorSubcoreMesh(core_axis_name="c", subcore_axis_name="s"),
    compiler_params=pltpu.CompilerParams(use_tc_tiling_on_sc=False,
                                         needs_layout_passes=False),
    scratch_types=[
        pltpu.VMEM((B_PER_W, D), jnp.float32),
        pltpu.VMEM((B_PER_W, D), jnp.float32),
        pltpu.VMEM((N_GCHUNKS, GCHUNK), jnp.int32),
        pltpu.VMEM((KP1, D), jnp.float32),
        pltpu.VMEM((KP1,), jnp.float32),
        pltpu.VMEM((B,), jnp.int32),
        pltpu.SMEM((B,), jnp.int32),
        pltpu.VMEM((COPY_CHUNK, D), jnp.float32),
        pltpu.VMEM((L,), jnp.float32),
        pltpu.SemaphoreType.DMA,
    ],
)(_sc_body)


def _tc_norm_body(e_ref, ps_ref, out_ref):
    z = jnp.sum(ps_ref[...]) * (float(N_ROWS) / float(B * KP1))
    out_ref[...] = e_ref[...] * (1.0 / z)


def _tc_norm(e, ps):
    return pl.pallas_call(
        _tc_norm_body,
        out_shape=jax.ShapeDtypeStruct((B, KP1), jnp.float32),
    )(e, ps)


def kernel(anchor_feature, pair_feature, memory_bank, negatives, membank_idx):
    e, ps, new_mb = _sc_call(
        anchor_feature, pair_feature, memory_bank,
        negatives.astype(jnp.int32), membank_idx.astype(jnp.int32))
    out_c = _tc_norm(e, ps)
    return out_c[:, :, None], new_mb


# speedup vs baseline: 1.6706x; 1.3080x over previous
import functools

import jax
import jax.numpy as jnp
from jax import lax
from jax.experimental import pallas as pl
from jax.experimental.pallas import tpu as pltpu
from jax.experimental.pallas import tpu_sc as plsc

B = 1024
D = 128
N_ROWS = 100000
KP1 = 512
T = 0.07

NW = 32
B_PER_W = B // NW
ROWS_PER_W = N_ROWS // NW
COPY_CHUNK = 125
N_COPY_CHUNKS = ROWS_PER_W // COPY_CHUNK
GCHUNK = 128
N_GCHUNKS = KP1 // GCHUNK
G_PER_CHUNK = GCHUNK // 16
L = 16
IDX_BITS = 17


def _sc_body(anchor_hbm, pair_hbm, mb_hbm, neg_hbm, mbidx_hbm,
             e_hbm, psum_hbm, newmb_hbm,
             anch_v, pair_v, neg_v, buf0_v, buf1_v, eblk_v, mbidx_v,
             hits_s, cbuf_v, psv_v, gsem0, gsem1, esem):
    c = lax.axis_index("c")
    s = lax.axis_index("s")
    w = s * 2 + c
    base_b = w * B_PER_W
    bufs = (buf0_v, buf1_v)
    gsems = (gsem0, gsem1)

    pltpu.sync_copy(anchor_hbm.at[pl.ds(base_b, B_PER_W)], anch_v)
    pltpu.sync_copy(pair_hbm.at[pl.ds(base_b, B_PER_W)], pair_v)
    pltpu.sync_copy(neg_hbm.at[pl.ds(base_b, B_PER_W)], neg_v)
    pltpu.sync_copy(mbidx_hbm, mbidx_v)

    iota = lax.iota(jnp.int32, L)

    def _gather(b, ci, par):
        return pltpu.make_async_copy(
            mb_hbm.at[neg_v.at[b, pl.ds(ci * GCHUNK, GCHUNK)]],
            bufs[par], gsems[par])

    _gather(0, 0, 0).start()

    def b_loop(b, psum):
        a = [anch_v[b, pl.ds(L * j, L)] for j in range(D // L)]
        for ci in range(N_GCHUNKS):
            par = ci % 2
            if ci < N_GCHUNKS - 1:
                _gather(b, ci + 1, (ci + 1) % 2).start()
            else:
                @pl.when(b < B_PER_W - 1)
                def _():
                    _gather(b + 1, 0, 0).start()
            _gather(b, ci, par).wait()
            buf = bufs[par]
            if ci == 0:
                for j in range(D // L):
                    buf[0, pl.ds(L * j, L)] = pair_v[b, pl.ds(L * j, L)]

            def g_loop(gg, psum):
                row_idx = iota + gg * L
                accs = [jnp.zeros((L,), jnp.float32) for _ in range(4)]
                for j in range(D // L):
                    for r in range(L):
                        d = L * j + r
                        ab = a[j].at[jnp.full((L,), r, jnp.int32)].get(
                            mode="promise_in_bounds")
                        v = plsc.load_gather(
                            buf, [row_idx, jnp.full((L,), d, jnp.int32)])
                        accs[d % 4] = accs[d % 4] + v * ab
                acc = (accs[0] + accs[1]) + (accs[2] + accs[3])
                e = jnp.exp(acc * (1.0 / T))
                eblk_v[b, pl.ds(ci * GCHUNK + gg * L, L)] = e
                return psum + e

            psum = lax.fori_loop(0, G_PER_CHUNK, g_loop, psum)
        return psum

    psum = lax.fori_loop(0, B_PER_W, b_loop, jnp.zeros((L,), jnp.float32))
    ecp = pltpu.make_async_copy(eblk_v, e_hbm.at[pl.ds(base_b, B_PER_W)],
                                esem)
    ecp.start()
    psv_v[...] = psum
    pltpu.sync_copy(psv_v, psum_hbm.at[w])

    lo = w * ROWS_PER_W

    def scan_b(t, cnt):
        vec = mbidx_v[pl.ds(L * t, L)]
        for u in range(L):
            idx = vec[u]
            b = L * t + u
            hit = (idx >= lo) & (idx < lo + ROWS_PER_W)

            @pl.when(hit)
            def _(cnt=cnt, b=b, idx=idx):
                hits_s[cnt] = (b << IDX_BITS) | idx

            cnt = cnt + jnp.where(hit, 1, 0)
        return cnt

    cnt = lax.fori_loop(0, B // L, scan_b, 0)

    def chunk_loop(ch, _):
        start = lo + ch * COPY_CHUNK
        pltpu.sync_copy(mb_hbm.at[pl.ds(start, COPY_CHUNK)], cbuf_v)

        def hit_loop(j, _):
            v = hits_s[j]
            hb = v >> IDX_BITS
            hi = v & ((1 << IDX_BITS) - 1)

            @pl.when((hi >= start) & (hi < start + COPY_CHUNK))
            def _():
                pltpu.sync_copy(anchor_hbm.at[hb], cbuf_v.at[hi - start])

            return 0

        lax.fori_loop(0, cnt, hit_loop, 0)
        pltpu.sync_copy(cbuf_v, newmb_hbm.at[pl.ds(start, COPY_CHUNK)])
        return 0

    lax.fori_loop(0, N_COPY_CHUNKS, chunk_loop, 0)
    ecp.wait()


_sc_call = functools.partial(
    pl.kernel,
    out_type=(
        jax.ShapeDtypeStruct((B, KP1), jnp.float32),
        jax.ShapeDtypeStruct((NW, L), jnp.float32),
        jax.ShapeDtypeStruct((N_ROWS, D), jnp.float32),
    ),
    mesh=plsc.VectorSubcoreMesh(core_axis_name="c", subcore_axis_name="s"),
    compiler_params=pltpu.CompilerParams(use_tc_tiling_on_sc=False,
                                         needs_layout_passes=False),
    scratch_types=[
        pltpu.VMEM((B_PER_W, D), jnp.float32),
        pltpu.VMEM((B_PER_W, D), jnp.float32),
        pltpu.VMEM((B_PER_W, KP1), jnp.int32),
        pltpu.VMEM((GCHUNK, D), jnp.float32),
        pltpu.VMEM((GCHUNK, D), jnp.float32),
        pltpu.VMEM((B_PER_W, KP1), jnp.float32),
        pltpu.VMEM((B,), jnp.int32),
        pltpu.SMEM((B,), jnp.int32),
        pltpu.VMEM((COPY_CHUNK, D), jnp.float32),
        pltpu.VMEM((L,), jnp.float32),
        pltpu.SemaphoreType.DMA,
        pltpu.SemaphoreType.DMA,
        pltpu.SemaphoreType.DMA,
    ],
)(_sc_body)


def _tc_norm_body(e_ref, ps_ref, out_ref):
    z = jnp.sum(ps_ref[...]) * (float(N_ROWS) / float(B * KP1))
    out_ref[...] = e_ref[...] * (1.0 / z)


def _tc_norm(e, ps):
    return pl.pallas_call(
        _tc_norm_body,
        out_shape=jax.ShapeDtypeStruct((B, KP1), jnp.float32),
    )(e, ps)


def kernel(anchor_feature, pair_feature, memory_bank, negatives, membank_idx):
    e, ps, new_mb = _sc_call(
        anchor_feature, pair_feature, memory_bank,
        negatives.astype(jnp.int32), membank_idx.astype(jnp.int32))
    out_c = _tc_norm(e, ps)
    return out_c[:, :, None], new_mb
